# baseline (speedup 1.0000x reference)
import functools

import numpy as np
import jax
import jax.numpy as jnp
from jax import lax
from jax.experimental import pallas as pl
from jax.experimental.pallas import tpu as pltpu
from jax.experimental.pallas import tpu_sc as plsc

_FEAT_STRIDE = 16
_PRE_NMS = 6000
_POST_NMS = 300
_NMS_THRESH = 0.7
_N = 9 * 64 * 64
_ROWS, _COLS = 8, 4608
_CN = 6144
_CCOLS = _CN // 8
_NEG = -jnp.inf


def _gen_anchors():
    base = np.array([1, 1, _FEAT_STRIDE, _FEAT_STRIDE], dtype=np.float64) - 1
    w = base[2] - base[0] + 1.0
    h = base[3] - base[1] + 1.0
    xc, yc = base[0] + 0.5 * (w - 1), base[1] + 0.5 * (h - 1)
    ratios = np.array([0.5, 1.0, 2.0])
    size = w * h
    ws_r = np.round(np.sqrt(size / ratios))
    hs_r = np.round(ws_r * ratios)
    anchors = []
    for wr, hr in zip(ws_r, hs_r):
        for s in (8.0, 16.0, 32.0):
            wss, hss = wr * s, hr * s
            anchors.append([xc - 0.5 * (wss - 1), yc - 0.5 * (hss - 1),
                            xc + 0.5 * (wss - 1), yc + 0.5 * (hss - 1)])
    return np.array(anchors, dtype=np.float64)


def _anchor_planes():
    a = _gen_anchors()
    sx = np.arange(64, dtype=np.float64) * _FEAT_STRIDE
    SX, SY = np.meshgrid(sx, sx)
    shift = np.stack([SX.ravel(), SY.ravel(), SX.ravel(), SY.ravel()], axis=1)
    full = (a[None, :, :] + shift[:, None, :]).reshape(_N, 4)
    x1, y1, x2, y2 = full[:, 0], full[:, 1], full[:, 2], full[:, 3]
    W = x2 - x1 + 1.0
    H = y2 - y1 + 1.0
    CX = x1 + 0.5 * W
    CY = y1 + 0.5 * H
    return [p.reshape(_ROWS, _COLS).astype(np.float32) for p in (W, H, CX, CY)]


_ANCHOR_PLANES = _anchor_planes()


def _cumsum_lanes(x, n):
    sh = 1
    r, c = x.shape
    while sh < n:
        x = x + jnp.concatenate(
            [jnp.zeros((r, sh), dtype=x.dtype), x[:, : c - sh]], axis=1)
        sh *= 2
    return x


def _cumsum_rows(x, n):
    sh = 1
    r, c = x.shape
    while sh < n:
        x = x + jnp.concatenate(
            [jnp.zeros((sh, c), dtype=x.dtype), x[: r - sh, :]], axis=0)
        sh *= 2
    return x


def _stage_a_kernel(sc_ref, dx_ref, dy_ref, dw_ref, dh_ref,
                    aw_ref, ah_ref, acx_ref, acy_ref, bnd_ref,
                    x1_ref, y1_ref, x2_ref, y2_ref, dst_ref):
    b = pl.program_id(0)

    W = aw_ref[...]
    H = ah_ref[...]
    pcx = dx_ref[0] * W + acx_ref[...]
    pcy = dy_ref[0] * H + acy_ref[...]
    pw = jnp.exp(dw_ref[0]) * W
    ph = jnp.exp(dh_ref[0]) * H
    mw = bnd_ref[0, 0] - 1.0
    mh = bnd_ref[0, 1] - 1.0
    x1_ref[0] = jnp.clip(pcx - 0.5 * pw, 0.0, mw)
    y1_ref[0] = jnp.clip(pcy - 0.5 * ph, 0.0, mh)
    x2_ref[0] = jnp.clip(pcx + 0.5 * pw, 0.0, mw)
    y2_ref[0] = jnp.clip(pcy + 0.5 * ph, 0.0, mh)

    s = sc_ref[0]
    si = lax.bitcast_convert_type(s, jnp.int32)
    keys = jnp.where(si < 0, si ^ jnp.int32(0x7FFFFFFF), si)

    sign = jnp.int32(-2147483648)
    cand = jnp.int32(0)
    for bbit in range(31, -1, -1):
        bit = jnp.int32(-(1 << 31)) if bbit == 31 else jnp.int32(1 << bbit)
        cand2 = cand | bit
        cnt = jnp.sum((keys >= (cand2 ^ sign)).astype(jnp.int32))
        cand = jnp.where(cnt >= _PRE_NMS, cand2, cand)
    Vs = cand ^ sign

    ri = lax.broadcasted_iota(jnp.int32, (_ROWS, _COLS), 0)
    ci = lax.broadcasted_iota(jnp.int32, (_ROWS, _COLS), 1)
    idx = ri * _COLS + ci

    c_gt = jnp.sum((keys > Vs).astype(jnp.int32))
    r = jnp.int32(_PRE_NMS) - c_gt
    eq = keys == Vs
    mc = jnp.int32(0)
    for bbit in range(16, -1, -1):
        cand2 = mc | jnp.int32(1 << bbit)
        g = jnp.sum((eq & (idx < cand2)).astype(jnp.int32))
        mc = jnp.where(g <= r, cand2, mc)
    mask = (keys > Vs) | (eq & (idx < mc))

    m32 = mask.astype(jnp.int32)
    c1 = _cumsum_lanes(m32, _COLS)
    rowtot = c1[:, _COLS - 1:_COLS]
    rowoff = _cumsum_rows(rowtot, _ROWS) - rowtot
    rank = c1 + rowoff
    base = b * _CN
    dst_ref[0] = jnp.where(mask, rank - 1 + base, jnp.int32(_PRE_NMS) + base)


def _stage_b_kernel(x1h, y1h, x2h, y2h, sch, dsth,
                    cx1h, cy1h, cx2h, cy2h, csch,
                    dst_v, pay_v, sem):
    nw = 32
    chunk = (2 * _N) // nw
    wid = lax.axis_index("s") * 2 + lax.axis_index("c")
    base = wid * chunk
    pltpu.sync_copy(dsth.at[pl.ds(base, chunk)], dst_v)
    for src, dst in ((x1h, cx1h), (y1h, cy1h), (x2h, cx2h), (y2h, cy2h),
                     (sch, csch)):
        pltpu.sync_copy(src.at[pl.ds(base, chunk)], pay_v)
        pltpu.async_copy(pay_v, dst.at[dst_v], sem).wait()


def _compact(x1f, y1f, x2f, y2f, scf, dstf, bsz):
    mesh = plsc.VectorSubcoreMesh(core_axis_name="c", subcore_axis_name="s")
    cf = jax.ShapeDtypeStruct((bsz * _CN,), jnp.float32)
    chunk = (bsz * _N) // 32
    sc_call = functools.partial(
        pl.kernel, mesh=mesh,
        out_type=[cf, cf, cf, cf, cf],
        scratch_types=[pltpu.VMEM((chunk,), jnp.int32),
                       pltpu.VMEM((chunk,), jnp.float32),
                       pltpu.SemaphoreType.DMA],
    )(_stage_b_kernel)
    return sc_call(x1f, y1f, x2f, y2f, scf, dstf)


def _stage_c_kernel(x1_ref, y1_ref, x2_ref, y2_ref, sc_ref, out_ref):
    b = pl.program_id(0)
    pos = lax.broadcasted_iota(jnp.int32, (_ROWS, _CCOLS), 0) * _CCOLS + \
          lax.broadcasted_iota(jnp.int32, (_ROWS, _CCOLS), 1)
    live = pos < _PRE_NMS
    x1 = x1_ref[0]
    y1 = y1_ref[0]
    x2 = x2_ref[0]
    y2 = y2_ref[0]
    ar = (x2 - x1 + 1.0) * (y2 - y1 + 1.0)
    ms0 = jnp.where(live, sc_ref[0], _NEG)

    si8 = lax.broadcasted_iota(jnp.int32, (_ROWS, 512), 0)
    li = lax.broadcasted_iota(jnp.int32, (_ROWS, 512), 1)
    out0 = jnp.where(si8 == 4, b.astype(jnp.float32), 0.0)

    def body(i, carry):
        ms, out = carry
        m = jnp.max(ms)
        valid = m != _NEG
        is_m = ms == m
        selpos = jnp.min(jnp.where(is_m, pos, jnp.int32(2147483647)))
        sel = is_m & (pos == selpos)
        x1s = jnp.sum(jnp.where(sel, x1, 0.0))
        y1s = jnp.sum(jnp.where(sel, y1, 0.0))
        x2s = jnp.sum(jnp.where(sel, x2, 0.0))
        y2s = jnp.sum(jnp.where(sel, y2, 0.0))
        ars = (x2s - x1s + 1.0) * (y2s - y1s + 1.0)
        xx1 = jnp.maximum(x1, x1s)
        yy1 = jnp.maximum(y1, y1s)
        xx2 = jnp.minimum(x2, x2s)
        yy2 = jnp.minimum(y2, y2s)
        w = jnp.maximum(xx2 - xx1 + 1.0, 0.0)
        h = jnp.maximum(yy2 - yy1 + 1.0, 0.0)
        inter = w * h
        sup = ((1.0 + _NMS_THRESH) * inter > _NMS_THRESH * (ars + ar)) & valid
        ms = jnp.where(sup, _NEG, ms)
        onehot = (li == i) & valid
        vals = jnp.where(si8 == 0, x1s,
               jnp.where(si8 == 1, y1s,
               jnp.where(si8 == 2, x2s, y2s)))
        out = out + jnp.where(onehot & (si8 < 4), vals, 0.0)
        return ms, out

    _, out = lax.fori_loop(0, _POST_NMS, body, (ms0, out0))
    out_ref[0] = out


def kernel(scores, bbox_deltas, image_width, image_height, is_training):
    bsz = scores.shape[0]
    na = 9
    sc = scores[:, na:, :, :].transpose(0, 2, 3, 1).reshape(bsz, _ROWS, _COLS)
    d = bbox_deltas.transpose(0, 2, 3, 1).reshape(bsz, _N, 4)
    dx = d[..., 0].reshape(bsz, _ROWS, _COLS)
    dy = d[..., 1].reshape(bsz, _ROWS, _COLS)
    dw = d[..., 2].reshape(bsz, _ROWS, _COLS)
    dh = d[..., 3].reshape(bsz, _ROWS, _COLS)
    bnd = jnp.stack([jnp.asarray(image_width, jnp.float32),
                     jnp.asarray(image_height, jnp.float32)]).reshape(1, 2)
    planes = [jnp.asarray(p) for p in _ANCHOR_PLANES]

    bspec = pl.BlockSpec((1, _ROWS, _COLS), lambda b: (b, 0, 0))
    cspec = pl.BlockSpec((_ROWS, _COLS), lambda b: (0, 0))
    fl = jax.ShapeDtypeStruct((bsz, _ROWS, _COLS), jnp.float32)
    x1f, y1f, x2f, y2f, dstf = pl.pallas_call(
        _stage_a_kernel,
        grid=(bsz,),
        in_specs=[bspec] * 5 + [cspec] * 4 + [pl.BlockSpec((1, 2), lambda b: (0, 0))],
        out_specs=[bspec] * 5,
        out_shape=[fl, fl, fl, fl,
                   jax.ShapeDtypeStruct((bsz, _ROWS, _COLS), jnp.int32)],
    )(sc, dx, dy, dw, dh, *planes, bnd)

    cx1, cy1, cx2, cy2, csc = _compact(
        x1f.reshape(-1), y1f.reshape(-1), x2f.reshape(-1), y2f.reshape(-1),
        sc.reshape(-1), dstf.reshape(-1), bsz)

    cbspec = pl.BlockSpec((1, _ROWS, _CCOLS), lambda b: (b, 0, 0))
    out = pl.pallas_call(
        _stage_c_kernel,
        grid=(bsz,),
        in_specs=[cbspec] * 5,
        out_specs=pl.BlockSpec((1, _ROWS, 512), lambda b: (b, 0, 0)),
        out_shape=jax.ShapeDtypeStruct((bsz, _ROWS, 512), jnp.float32),
    )(cx1.reshape(bsz, _ROWS, _CCOLS), cy1.reshape(bsz, _ROWS, _CCOLS),
      cx2.reshape(bsz, _ROWS, _CCOLS), cy2.reshape(bsz, _ROWS, _CCOLS),
      csc.reshape(bsz, _ROWS, _CCOLS))

    coords = out[:, 0:4, :_POST_NMS]
    col0 = out[:, 4:5, :_POST_NMS]
    return jnp.concatenate([col0, coords], axis=1).transpose(0, 2, 1)


# speedup vs baseline: 2.3154x; 2.3154x over previous
import functools

import numpy as np
import jax
import jax.numpy as jnp
from jax import lax
from jax.experimental import pallas as pl
from jax.experimental.pallas import tpu as pltpu
from jax.experimental.pallas import tpu_sc as plsc

_FEAT_STRIDE = 16
_PRE_NMS = 6000
_POST_NMS = 300
_NMS_THRESH = 0.7
_N = 9 * 64 * 64
_ROWS, _COLS = 8, 4608
_CN = 6144
_CCOLS = _CN // 8
_NEG = -jnp.inf


def _gen_anchors():
    base = np.array([1, 1, _FEAT_STRIDE, _FEAT_STRIDE], dtype=np.float64) - 1
    w = base[2] - base[0] + 1.0
    h = base[3] - base[1] + 1.0
    xc, yc = base[0] + 0.5 * (w - 1), base[1] + 0.5 * (h - 1)
    ratios = np.array([0.5, 1.0, 2.0])
    size = w * h
    ws_r = np.round(np.sqrt(size / ratios))
    hs_r = np.round(ws_r * ratios)
    anchors = []
    for wr, hr in zip(ws_r, hs_r):
        for s in (8.0, 16.0, 32.0):
            wss, hss = wr * s, hr * s
            anchors.append([xc - 0.5 * (wss - 1), yc - 0.5 * (hss - 1),
                            xc + 0.5 * (wss - 1), yc + 0.5 * (hss - 1)])
    return np.array(anchors, dtype=np.float64)


def _anchor_planes():
    a = _gen_anchors()
    sx = np.arange(64, dtype=np.float64) * _FEAT_STRIDE
    SX, SY = np.meshgrid(sx, sx)
    shift = np.stack([SX.ravel(), SY.ravel(), SX.ravel(), SY.ravel()], axis=1)
    full = (a[None, :, :] + shift[:, None, :]).reshape(_N, 4)
    x1, y1, x2, y2 = full[:, 0], full[:, 1], full[:, 2], full[:, 3]
    W = x2 - x1 + 1.0
    H = y2 - y1 + 1.0
    CX = x1 + 0.5 * W
    CY = y1 + 0.5 * H
    return [p.reshape(_ROWS, _COLS).astype(np.float32) for p in (W, H, CX, CY)]


_ANCHOR_PLANES = _anchor_planes()


def _cumsum_lanes(x, n):
    sh = 1
    r, c = x.shape
    while sh < n:
        x = x + jnp.concatenate(
            [jnp.zeros((r, sh), dtype=x.dtype), x[:, : c - sh]], axis=1)
        sh *= 2
    return x


def _cumsum_rows(x, n):
    sh = 1
    r, c = x.shape
    while sh < n:
        x = x + jnp.concatenate(
            [jnp.zeros((sh, c), dtype=x.dtype), x[: r - sh, :]], axis=0)
        sh *= 2
    return x


def _stage_a_kernel(sc_ref, dx_ref, dy_ref, dw_ref, dh_ref,
                    aw_ref, ah_ref, acx_ref, acy_ref, bnd_ref,
                    x1_ref, y1_ref, x2_ref, y2_ref, dst_ref):
    b = pl.program_id(0)

    W = aw_ref[...]
    H = ah_ref[...]
    pcx = dx_ref[0] * W + acx_ref[...]
    pcy = dy_ref[0] * H + acy_ref[...]
    pw = jnp.exp(dw_ref[0]) * W
    ph = jnp.exp(dh_ref[0]) * H
    mw = bnd_ref[0, 0] - 1.0
    mh = bnd_ref[0, 1] - 1.0
    x1_ref[0] = jnp.clip(pcx - 0.5 * pw, 0.0, mw)
    y1_ref[0] = jnp.clip(pcy - 0.5 * ph, 0.0, mh)
    x2_ref[0] = jnp.clip(pcx + 0.5 * pw, 0.0, mw)
    y2_ref[0] = jnp.clip(pcy + 0.5 * ph, 0.0, mh)

    s = sc_ref[0]
    si = lax.bitcast_convert_type(s, jnp.int32)
    keys = jnp.where(si < 0, si ^ jnp.int32(0x7FFFFFFF), si)

    sign = jnp.int32(-2147483648)
    cand = jnp.int32(0)
    for bbit in range(31, -1, -1):
        bit = jnp.int32(-(1 << 31)) if bbit == 31 else jnp.int32(1 << bbit)
        cand2 = cand | bit
        cnt = jnp.sum((keys >= (cand2 ^ sign)).astype(jnp.int32))
        cand = jnp.where(cnt >= _PRE_NMS, cand2, cand)
    Vs = cand ^ sign

    ri = lax.broadcasted_iota(jnp.int32, (_ROWS, _COLS), 0)
    ci = lax.broadcasted_iota(jnp.int32, (_ROWS, _COLS), 1)
    idx = ri * _COLS + ci

    c_gt = jnp.sum((keys > Vs).astype(jnp.int32))
    r = jnp.int32(_PRE_NMS) - c_gt
    eq = keys == Vs
    mc = jnp.int32(0)
    for bbit in range(16, -1, -1):
        cand2 = mc | jnp.int32(1 << bbit)
        g = jnp.sum((eq & (idx < cand2)).astype(jnp.int32))
        mc = jnp.where(g <= r, cand2, mc)
    mask = (keys > Vs) | (eq & (idx < mc))

    m32 = mask.astype(jnp.int32)
    c1 = _cumsum_lanes(m32, _COLS)
    rowtot = c1[:, _COLS - 1:_COLS]
    rowoff = _cumsum_rows(rowtot, _ROWS) - rowtot
    rank = c1 + rowoff
    base = b * _CN
    dst_ref[0] = jnp.where(mask, rank - 1 + base, jnp.int32(_PRE_NMS) + base)


def _stage_b_kernel(rows, x1h, y1h, x2h, y2h, sch, dsth,
                    cx1h, cy1h, cx2h, cy2h, csch,
                    dst_v, p0, p1, p2, p3, p4, sem):
    wid = lax.axis_index("s") * 2 + lax.axis_index("c")
    pltpu.sync_copy(dsth.at[wid], dst_v)
    pairs = ((x1h, p0, cx1h), (y1h, p1, cy1h), (x2h, p2, cx2h),
             (y2h, p3, cy2h), (sch, p4, csch))
    for src, pv, _ in pairs:
        pltpu.sync_copy(src.at[wid], pv)

    @pl.loop(0, rows)
    def _scatter(j):
        copies = [pltpu.async_copy(pv.at[j], dst.at[dst_v.at[j]], sem)
                  for _, pv, dst in pairs]
        for c in copies:
            c.wait()


def _compact(x1f, y1f, x2f, y2f, scf, dstf, bsz):
    mesh = plsc.VectorSubcoreMesh(core_axis_name="c", subcore_axis_name="s")
    cf = jax.ShapeDtypeStruct((bsz * _CN,), jnp.float32)
    rows = (bsz * _N) // (32 * 128)
    r3 = lambda a: a.reshape(32, rows, 128)
    sc_call = functools.partial(
        pl.kernel, mesh=mesh,
        out_type=[cf, cf, cf, cf, cf],
        scratch_types=[pltpu.VMEM((rows, 128), jnp.int32)]
                      + [pltpu.VMEM((rows, 128), jnp.float32)] * 5
                      + [pltpu.SemaphoreType.DMA],
    )(functools.partial(_stage_b_kernel, rows))
    return sc_call(r3(x1f), r3(y1f), r3(x2f), r3(y2f), r3(scf), r3(dstf))


def _stage_c_kernel(x1_ref, y1_ref, x2_ref, y2_ref, sc_ref, out_ref):
    b = pl.program_id(0)
    pos = lax.broadcasted_iota(jnp.int32, (_ROWS, _CCOLS), 0) * _CCOLS + \
          lax.broadcasted_iota(jnp.int32, (_ROWS, _CCOLS), 1)
    live = pos < _PRE_NMS
    x1 = x1_ref[0]
    y1 = y1_ref[0]
    x2 = x2_ref[0]
    y2 = y2_ref[0]
    ar = (x2 - x1 + 1.0) * (y2 - y1 + 1.0)
    ms0 = jnp.where(live, sc_ref[0], _NEG)

    si8 = lax.broadcasted_iota(jnp.int32, (_ROWS, 512), 0)
    li = lax.broadcasted_iota(jnp.int32, (_ROWS, 512), 1)
    out0 = jnp.where(si8 == 4, b.astype(jnp.float32), 0.0)

    def body(i, carry):
        ms, out = carry
        m = jnp.max(ms)
        valid = m != _NEG
        is_m = ms == m
        selpos = jnp.min(jnp.where(is_m, pos, jnp.int32(2147483647)))
        sel = is_m & (pos == selpos)
        x1s = jnp.sum(jnp.where(sel, x1, 0.0))
        y1s = jnp.sum(jnp.where(sel, y1, 0.0))
        x2s = jnp.sum(jnp.where(sel, x2, 0.0))
        y2s = jnp.sum(jnp.where(sel, y2, 0.0))
        ars = (x2s - x1s + 1.0) * (y2s - y1s + 1.0)
        xx1 = jnp.maximum(x1, x1s)
        yy1 = jnp.maximum(y1, y1s)
        xx2 = jnp.minimum(x2, x2s)
        yy2 = jnp.minimum(y2, y2s)
        w = jnp.maximum(xx2 - xx1 + 1.0, 0.0)
        h = jnp.maximum(yy2 - yy1 + 1.0, 0.0)
        inter = w * h
        sup = ((1.0 + _NMS_THRESH) * inter > _NMS_THRESH * (ars + ar)) & valid
        ms = jnp.where(sup, _NEG, ms)
        onehot = (li == i) & valid
        vals = jnp.where(si8 == 0, x1s,
               jnp.where(si8 == 1, y1s,
               jnp.where(si8 == 2, x2s, y2s)))
        out = out + jnp.where(onehot & (si8 < 4), vals, 0.0)
        return ms, out

    _, out = lax.fori_loop(0, _POST_NMS, body, (ms0, out0))
    out_ref[0] = out


def kernel(scores, bbox_deltas, image_width, image_height, is_training):
    bsz = scores.shape[0]
    na = 9
    sc = scores[:, na:, :, :].transpose(0, 2, 3, 1).reshape(bsz, _ROWS, _COLS)
    d = bbox_deltas.transpose(0, 2, 3, 1).reshape(bsz, _N, 4)
    dx = d[..., 0].reshape(bsz, _ROWS, _COLS)
    dy = d[..., 1].reshape(bsz, _ROWS, _COLS)
    dw = d[..., 2].reshape(bsz, _ROWS, _COLS)
    dh = d[..., 3].reshape(bsz, _ROWS, _COLS)
    bnd = jnp.stack([jnp.asarray(image_width, jnp.float32),
                     jnp.asarray(image_height, jnp.float32)]).reshape(1, 2)
    planes = [jnp.asarray(p) for p in _ANCHOR_PLANES]

    bspec = pl.BlockSpec((1, _ROWS, _COLS), lambda b: (b, 0, 0))
    cspec = pl.BlockSpec((_ROWS, _COLS), lambda b: (0, 0))
    fl = jax.ShapeDtypeStruct((bsz, _ROWS, _COLS), jnp.float32)
    x1f, y1f, x2f, y2f, dstf = pl.pallas_call(
        _stage_a_kernel,
        grid=(bsz,),
        in_specs=[bspec] * 5 + [cspec] * 4 + [pl.BlockSpec((1, 2), lambda b: (0, 0))],
        out_specs=[bspec] * 5,
        out_shape=[fl, fl, fl, fl,
                   jax.ShapeDtypeStruct((bsz, _ROWS, _COLS), jnp.int32)],
    )(sc, dx, dy, dw, dh, *planes, bnd)

    cx1, cy1, cx2, cy2, csc = _compact(
        x1f.reshape(-1), y1f.reshape(-1), x2f.reshape(-1), y2f.reshape(-1),
        sc.reshape(-1), dstf.reshape(-1), bsz)

    cbspec = pl.BlockSpec((1, _ROWS, _CCOLS), lambda b: (b, 0, 0))
    out = pl.pallas_call(
        _stage_c_kernel,
        grid=(bsz,),
        in_specs=[cbspec] * 5,
        out_specs=pl.BlockSpec((1, _ROWS, 512), lambda b: (b, 0, 0)),
        out_shape=jax.ShapeDtypeStruct((bsz, _ROWS, 512), jnp.float32),
    )(cx1.reshape(bsz, _ROWS, _CCOLS), cy1.reshape(bsz, _ROWS, _CCOLS),
      cx2.reshape(bsz, _ROWS, _CCOLS), cy2.reshape(bsz, _ROWS, _CCOLS),
      csc.reshape(bsz, _ROWS, _CCOLS))

    coords = out[:, 0:4, :_POST_NMS]
    col0 = out[:, 4:5, :_POST_NMS]
    return jnp.concatenate([col0, coords], axis=1).transpose(0, 2, 1)


# speedup vs baseline: 14.7020x; 6.3498x over previous
import functools

import numpy as np
import jax
import jax.numpy as jnp
from jax import lax
from jax.experimental import pallas as pl
from jax.experimental.pallas import tpu as pltpu
from jax.experimental.pallas import tpu_sc as plsc

_FEAT_STRIDE = 16
_PRE_NMS = 6000
_POST_NMS = 300
_NMS_THRESH = 0.7
_N = 9 * 64 * 64
_ROWS, _COLS = 8, 4608
_CN = 6144
_CCOLS = _CN // 8
_SLAB = _CN + _N
_NEG = -jnp.inf


def _gen_anchors():
    base = np.array([1, 1, _FEAT_STRIDE, _FEAT_STRIDE], dtype=np.float64) - 1
    w = base[2] - base[0] + 1.0
    h = base[3] - base[1] + 1.0
    xc, yc = base[0] + 0.5 * (w - 1), base[1] + 0.5 * (h - 1)
    ratios = np.array([0.5, 1.0, 2.0])
    size = w * h
    ws_r = np.round(np.sqrt(size / ratios))
    hs_r = np.round(ws_r * ratios)
    anchors = []
    for wr, hr in zip(ws_r, hs_r):
        for s in (8.0, 16.0, 32.0):
            wss, hss = wr * s, hr * s
            anchors.append([xc - 0.5 * (wss - 1), yc - 0.5 * (hss - 1),
                            xc + 0.5 * (wss - 1), yc + 0.5 * (hss - 1)])
    return np.array(anchors, dtype=np.float64)


def _anchor_planes():
    a = _gen_anchors()
    sx = np.arange(64, dtype=np.float64) * _FEAT_STRIDE
    SX, SY = np.meshgrid(sx, sx)
    shift = np.stack([SX.ravel(), SY.ravel(), SX.ravel(), SY.ravel()], axis=1)
    full = (a[None, :, :] + shift[:, None, :]).reshape(_N, 4)
    x1, y1, x2, y2 = full[:, 0], full[:, 1], full[:, 2], full[:, 3]
    W = x2 - x1 + 1.0
    H = y2 - y1 + 1.0
    CX = x1 + 0.5 * W
    CY = y1 + 0.5 * H
    return [p.reshape(_ROWS, _COLS).astype(np.float32) for p in (W, H, CX, CY)]


_ANCHOR_PLANES = _anchor_planes()


def _cumsum_lanes(x, n):
    sh = 1
    r, c = x.shape
    while sh < n:
        x = x + jnp.concatenate(
            [jnp.zeros((r, sh), dtype=x.dtype), x[:, : c - sh]], axis=1)
        sh *= 2
    return x


def _cumsum_rows(x, n):
    sh = 1
    r, c = x.shape
    while sh < n:
        x = x + jnp.concatenate(
            [jnp.zeros((sh, c), dtype=x.dtype), x[: r - sh, :]], axis=0)
        sh *= 2
    return x


def _stage_a_kernel(sc_ref, dx_ref, dy_ref, dw_ref, dh_ref,
                    aw_ref, ah_ref, acx_ref, acy_ref, bnd_ref,
                    x1_ref, y1_ref, x2_ref, y2_ref, dst_ref):
    b = pl.program_id(0)

    W = aw_ref[...]
    H = ah_ref[...]
    pcx = dx_ref[0] * W + acx_ref[...]
    pcy = dy_ref[0] * H + acy_ref[...]
    pw = jnp.exp(dw_ref[0]) * W
    ph = jnp.exp(dh_ref[0]) * H
    mw = bnd_ref[0, 0] - 1.0
    mh = bnd_ref[0, 1] - 1.0
    x1_ref[0] = jnp.clip(pcx - 0.5 * pw, 0.0, mw)
    y1_ref[0] = jnp.clip(pcy - 0.5 * ph, 0.0, mh)
    x2_ref[0] = jnp.clip(pcx + 0.5 * pw, 0.0, mw)
    y2_ref[0] = jnp.clip(pcy + 0.5 * ph, 0.0, mh)

    s = sc_ref[0]
    si = lax.bitcast_convert_type(s, jnp.int32)
    keys = jnp.where(si < 0, si ^ jnp.int32(0x7FFFFFFF), si)

    sign = jnp.int32(-2147483648)
    cand = jnp.int32(0)
    for bbit in range(31, -1, -1):
        bit = jnp.int32(-(1 << 31)) if bbit == 31 else jnp.int32(1 << bbit)
        cand2 = cand | bit
        cnt = jnp.sum((keys >= (cand2 ^ sign)).astype(jnp.int32))
        cand = jnp.where(cnt >= _PRE_NMS, cand2, cand)
    Vs = cand ^ sign

    ri = lax.broadcasted_iota(jnp.int32, (_ROWS, _COLS), 0)
    ci = lax.broadcasted_iota(jnp.int32, (_ROWS, _COLS), 1)
    idx = ri * _COLS + ci

    c_gt = jnp.sum((keys > Vs).astype(jnp.int32))
    r = jnp.int32(_PRE_NMS) - c_gt
    eq = keys == Vs
    mc = jnp.int32(0)
    for bbit in range(16, -1, -1):
        cand2 = mc | jnp.int32(1 << bbit)
        g = jnp.sum((eq & (idx < cand2)).astype(jnp.int32))
        mc = jnp.where(g <= r, cand2, mc)
    mask = (keys > Vs) | (eq & (idx < mc))

    m32 = mask.astype(jnp.int32)
    c1 = _cumsum_lanes(m32, _COLS)
    rowtot = c1[:, _COLS - 1:_COLS]
    rowoff = _cumsum_rows(rowtot, _ROWS) - rowtot
    rank = c1 + rowoff
    base = b * _SLAB
    dst_ref[0] = jnp.where(mask, rank - 1 + base,
                           jnp.int32(_CN) + (idx - rank) + base)


def _stage_b_kernel(rows, x1h, y1h, x2h, y2h, sch, dsth,
                    cx1h, cy1h, cx2h, cy2h, csch,
                    dst_v, p0, p1, p2, p3, p4, sem):
    wid = lax.axis_index("s") * 2 + lax.axis_index("c")
    pltpu.sync_copy(dsth.at[wid], dst_v)
    pairs = ((x1h, p0, cx1h), (y1h, p1, cy1h), (x2h, p2, cx2h),
             (y2h, p3, cy2h), (sch, p4, csch))
    for src, pv, _ in pairs:
        pltpu.sync_copy(src.at[wid], pv)

    @pl.loop(0, rows)
    def _scatter(j):
        copies = [pltpu.async_copy(pv.at[j], dst.at[dst_v.at[j]], sem)
                  for _, pv, dst in pairs]
        for c in copies:
            c.wait()


def _compact(x1f, y1f, x2f, y2f, scf, dstf, bsz):
    mesh = plsc.VectorSubcoreMesh(core_axis_name="c", subcore_axis_name="s")
    cf = jax.ShapeDtypeStruct((bsz * _SLAB,), jnp.float32)
    rows = (bsz * _N) // (32 * 128)
    r3 = lambda a: a.reshape(32, rows, 128)
    sc_call = functools.partial(
        pl.kernel, mesh=mesh,
        out_type=[cf, cf, cf, cf, cf],
        scratch_types=[pltpu.VMEM((rows, 128), jnp.int32)]
                      + [pltpu.VMEM((rows, 128), jnp.float32)] * 5
                      + [pltpu.SemaphoreType.DMA],
    )(functools.partial(_stage_b_kernel, rows))
    return sc_call(r3(x1f), r3(y1f), r3(x2f), r3(y2f), r3(scf), r3(dstf))


def _stage_c_kernel(x1_ref, y1_ref, x2_ref, y2_ref, sc_ref, out_ref):
    b = pl.program_id(0)
    pos = lax.broadcasted_iota(jnp.int32, (_ROWS, _CCOLS), 0) * _CCOLS + \
          lax.broadcasted_iota(jnp.int32, (_ROWS, _CCOLS), 1)
    live = pos < _PRE_NMS
    x1 = x1_ref[0]
    y1 = y1_ref[0]
    x2 = x2_ref[0]
    y2 = y2_ref[0]
    ar = (x2 - x1 + 1.0) * (y2 - y1 + 1.0)
    ms0 = jnp.where(live, sc_ref[0], _NEG)

    si8 = lax.broadcasted_iota(jnp.int32, (_ROWS, 512), 0)
    li = lax.broadcasted_iota(jnp.int32, (_ROWS, 512), 1)
    out0 = jnp.where(si8 == 4, b.astype(jnp.float32), 0.0)

    def body(i, carry):
        ms, out = carry
        m = jnp.max(ms)
        valid = m != _NEG
        is_m = ms == m
        selpos = jnp.min(jnp.where(is_m, pos, jnp.int32(2147483647)))
        sel = is_m & (pos == selpos)
        x1s = jnp.sum(jnp.where(sel, x1, 0.0))
        y1s = jnp.sum(jnp.where(sel, y1, 0.0))
        x2s = jnp.sum(jnp.where(sel, x2, 0.0))
        y2s = jnp.sum(jnp.where(sel, y2, 0.0))
        ars = (x2s - x1s + 1.0) * (y2s - y1s + 1.0)
        xx1 = jnp.maximum(x1, x1s)
        yy1 = jnp.maximum(y1, y1s)
        xx2 = jnp.minimum(x2, x2s)
        yy2 = jnp.minimum(y2, y2s)
        w = jnp.maximum(xx2 - xx1 + 1.0, 0.0)
        h = jnp.maximum(yy2 - yy1 + 1.0, 0.0)
        inter = w * h
        sup = ((1.0 + _NMS_THRESH) * inter > _NMS_THRESH * (ars + ar)) & valid
        ms = jnp.where(sup, _NEG, ms)
        onehot = (li == i) & valid
        vals = jnp.where(si8 == 0, x1s,
               jnp.where(si8 == 1, y1s,
               jnp.where(si8 == 2, x2s, y2s)))
        out = out + jnp.where(onehot & (si8 < 4), vals, 0.0)
        return ms, out

    _, out = lax.fori_loop(0, _POST_NMS, body, (ms0, out0))
    out_ref[0] = out


def kernel(scores, bbox_deltas, image_width, image_height, is_training):
    bsz = scores.shape[0]
    na = 9
    sc = scores[:, na:, :, :].transpose(0, 2, 3, 1).reshape(bsz, _ROWS, _COLS)
    d = bbox_deltas.transpose(0, 2, 3, 1).reshape(bsz, _N, 4)
    dx = d[..., 0].reshape(bsz, _ROWS, _COLS)
    dy = d[..., 1].reshape(bsz, _ROWS, _COLS)
    dw = d[..., 2].reshape(bsz, _ROWS, _COLS)
    dh = d[..., 3].reshape(bsz, _ROWS, _COLS)
    bnd = jnp.stack([jnp.asarray(image_width, jnp.float32),
                     jnp.asarray(image_height, jnp.float32)]).reshape(1, 2)
    planes = [jnp.asarray(p) for p in _ANCHOR_PLANES]

    bspec = pl.BlockSpec((1, _ROWS, _COLS), lambda b: (b, 0, 0))
    cspec = pl.BlockSpec((_ROWS, _COLS), lambda b: (0, 0))
    fl = jax.ShapeDtypeStruct((bsz, _ROWS, _COLS), jnp.float32)
    x1f, y1f, x2f, y2f, dstf = pl.pallas_call(
        _stage_a_kernel,
        grid=(bsz,),
        in_specs=[bspec] * 5 + [cspec] * 4 + [pl.BlockSpec((1, 2), lambda b: (0, 0))],
        out_specs=[bspec] * 5,
        out_shape=[fl, fl, fl, fl,
                   jax.ShapeDtypeStruct((bsz, _ROWS, _COLS), jnp.int32)],
    )(sc, dx, dy, dw, dh, *planes, bnd)

    cx1, cy1, cx2, cy2, csc = _compact(
        x1f.reshape(-1), y1f.reshape(-1), x2f.reshape(-1), y2f.reshape(-1),
        sc.reshape(-1), dstf.reshape(-1), bsz)

    trim = lambda a: a.reshape(bsz, _SLAB)[:, :_CN].reshape(bsz, _ROWS, _CCOLS)
    cbspec = pl.BlockSpec((1, _ROWS, _CCOLS), lambda b: (b, 0, 0))
    out = pl.pallas_call(
        _stage_c_kernel,
        grid=(bsz,),
        in_specs=[cbspec] * 5,
        out_specs=pl.BlockSpec((1, _ROWS, 512), lambda b: (b, 0, 0)),
        out_shape=jax.ShapeDtypeStruct((bsz, _ROWS, 512), jnp.float32),
    )(trim(cx1), trim(cy1), trim(cx2), trim(cy2), trim(csc))

    coords = out[:, 0:4, :_POST_NMS]
    col0 = out[:, 4:5, :_POST_NMS]
    return jnp.concatenate([col0, coords], axis=1).transpose(0, 2, 1)
